# baseline (speedup 1.0000x reference)
import functools

import jax
import jax.numpy as jnp
from jax import lax
from jax.experimental import pallas as pl
from jax.experimental.pallas import tpu as pltpu
from jax.experimental.pallas import tpu_sc as plsc

BATCH = 1024
VOCAB = 100000
PADDING_IDX = 0

NWORKERS = 32
ROWS_PER_WORKER = BATCH // NWORKERS
SPAN = ROWS_PER_WORKER * VOCAB
ZCHUNK = 12800
NZDMA = SPAN // ZCHUNK
LANES = 16


def _sc_body(tgt_hbm, out_hbm, zbuf, ids_v, idx_v, val_v, zsem, ssem):
    wid = lax.axis_index("c") * (NWORKERS // 2) + lax.axis_index("s")
    row0 = wid * ROWS_PER_WORKER
    start = wid * SPAN

    pltpu.sync_copy(tgt_hbm.at[pl.ds(row0, ROWS_PER_WORKER)], ids_v)
    for c in range(ROWS_PER_WORKER // LANES):
        ids_c = ids_v[pl.ds(c * LANES, LANES)]
        rows_c = row0 + c * LANES + lax.iota(jnp.int32, LANES)
        idx_v[pl.ds(c * LANES, LANES)] = rows_c * VOCAB + ids_c
        val_v[pl.ds(c * LANES, LANES)] = jnp.where(
            ids_c != PADDING_IDX, 1.0, 0.0
        ).astype(jnp.float32)

    def _zero_init(i, _):
        zbuf[pl.ds(pl.multiple_of(i * LANES, LANES), LANES)] = jnp.zeros(
            (LANES,), jnp.float32
        )
        return _

    lax.fori_loop(0, ZCHUNK // LANES, _zero_init, 0)

    def _fire(i, _):
        pltpu.make_async_copy(
            zbuf,
            out_hbm.at[pl.ds(start + i * ZCHUNK, ZCHUNK)],
            zsem,
        ).start()
        return _

    lax.fori_loop(0, NZDMA, _fire, 0)

    def _drain(i, _):
        pltpu.make_async_copy(
            zbuf,
            out_hbm.at[pl.ds(start + i * ZCHUNK, ZCHUNK)],
            zsem,
        ).wait()
        return _

    lax.fori_loop(0, NZDMA, _drain, 0)

    pltpu.make_async_copy(val_v, out_hbm.at[idx_v], ssem).start()
    pltpu.make_async_copy(val_v, out_hbm.at[idx_v], ssem).wait()


@jax.jit
def kernel(tgt_token_ids_batch):
    tgt = tgt_token_ids_batch.astype(jnp.int32).reshape(BATCH)
    sc_kernel = functools.partial(
        pl.kernel,
        out_type=jax.ShapeDtypeStruct((BATCH * VOCAB,), jnp.float32),
        mesh=plsc.VectorSubcoreMesh(core_axis_name="c", subcore_axis_name="s"),
        scratch_types=[
            pltpu.VMEM((ZCHUNK,), jnp.float32),
            pltpu.VMEM((ROWS_PER_WORKER,), jnp.int32),
            pltpu.VMEM((ROWS_PER_WORKER,), jnp.int32),
            pltpu.VMEM((ROWS_PER_WORKER,), jnp.float32),
            pltpu.SemaphoreType.DMA,
            pltpu.SemaphoreType.DMA,
        ],
    )(_sc_body)
    flat = sc_kernel(tgt)
    return flat.reshape(BATCH, VOCAB)
